# Initial kernel scaffold; baseline (speedup 1.0000x reference)
import jax
import jax.numpy as jnp
from jax.experimental import pallas as pl


def kernel(x, edge_index, W1l, W1r, b1, W2l, W2r, b2, Wout, bout):
    raise NotImplementedError("write your pallas kernel here")


# speedup vs baseline: 10.4911x; 10.4911x over previous
import functools

import jax
import jax.numpy as jnp
from jax import lax
from jax.experimental import pallas as pl
from jax.experimental.pallas import tpu as pltpu
from jax.experimental.pallas import tpu_sc as plsc

N = 10000
D = 128
H = 64

NC = 2
NS = 16
CH = 128
CPT = 80
NCHUNK = NC * NS * CPT
E_PAD = NCHUNK * CH
N_ACC = 10240
RPT = N_ACC // NS

_vmesh = plsc.VectorSubcoreMesh(core_axis_name="c", subcore_axis_name="s")
_sc_params = pltpu.CompilerParams(use_tc_tiling_on_sc=False)


@functools.partial(
    pl.kernel,
    out_type=jax.ShapeDtypeStruct((NC, N_ACC, H), jnp.float32),
    mesh=_vmesh,
    scratch_types=[
        pltpu.VMEM((CPT, CH), jnp.int32),
        pltpu.VMEM((CPT, CH), jnp.int32),
        pltpu.VMEM((CH, H), jnp.float32),
        pltpu.VMEM_SHARED((N_ACC, H), jnp.float32),
        pltpu.SemaphoreType.DMA,
    ],
    compiler_params=_sc_params,
)
def _sc_segsum(table_hbm, srcp_hbm, dstp_hbm, out_hbm,
               sidx, didx, rows, acc, sem):
    c = lax.axis_index("c")
    s = lax.axis_index("s")
    base_chunk = c * (NS * CPT) + s * CPT

    pltpu.sync_copy(srcp_hbm.at[pl.ds(base_chunk, CPT)], sidx)
    pltpu.sync_copy(dstp_hbm.at[pl.ds(base_chunk, CPT)], didx)

    @pl.loop(0, CH)
    def _(i):
        @pl.loop(0, H // 16)
        def _(j):
            rows[i, pl.ds(j * 16, 16)] = jnp.zeros((16,), jnp.float32)

    @pl.loop(0, RPT // CH)
    def _(j):
        pltpu.sync_copy(rows, acc.at[pl.ds(s * RPT + j * CH, CH)])

    plsc.subcore_barrier()

    @pl.loop(0, CPT)
    def _(i):
        pltpu.async_copy(table_hbm.at[sidx.at[i]], rows, sem).wait()
        pltpu.sync_copy(rows, acc.at[didx.at[i]], add=True)

    plsc.subcore_barrier()

    pltpu.sync_copy(acc.at[pl.ds(s * RPT, RPT)],
                    out_hbm.at[c].at[pl.ds(s * RPT, RPT)])


@functools.partial(
    pl.kernel,
    out_type=jax.ShapeDtypeStruct((NC, N_ACC, 16), jnp.float32),
    mesh=_vmesh,
    scratch_types=[
        pltpu.VMEM((CPT, CH), jnp.int32),
        pltpu.VMEM((CH, 16), jnp.float32),
        pltpu.VMEM_SHARED((N_ACC, 16), jnp.float32),
    ],
    compiler_params=_sc_params,
)
def _sc_degree(dstp_hbm, out_hbm, didx, ones, acc):
    c = lax.axis_index("c")
    s = lax.axis_index("s")
    base_chunk = c * (NS * CPT) + s * CPT

    pltpu.sync_copy(dstp_hbm.at[pl.ds(base_chunk, CPT)], didx)

    @pl.loop(0, CH)
    def _(i):
        ones[i, :] = jnp.zeros((16,), jnp.float32)

    @pl.loop(0, RPT // CH)
    def _(j):
        pltpu.sync_copy(ones, acc.at[pl.ds(s * RPT + j * CH, CH)])

    @pl.loop(0, CH)
    def _(i):
        ones[i, :] = jnp.ones((16,), jnp.float32)

    plsc.subcore_barrier()

    @pl.loop(0, CPT)
    def _(i):
        pltpu.sync_copy(ones, acc.at[didx.at[i]], add=True)

    plsc.subcore_barrier()

    pltpu.sync_copy(acc.at[pl.ds(s * RPT, RPT)],
                    out_hbm.at[c].at[pl.ds(s * RPT, RPT)])


_BM = 1000
_GRID = N // _BM


def _k1_body(x_ref, w_ref, b_ref, z_ref, xr_ref):
    o = jnp.dot(x_ref[...], w_ref[...], preferred_element_type=jnp.float32)
    z_ref[...] = o[:, :H]
    xr_ref[...] = o[:, H:] + b_ref[...]


def _k2_body(p_ref, dg_ref, xr_ref, w_ref, b_ref, z_ref, xr2_ref):
    deg = dg_ref[0, :, :1] + dg_ref[1, :, :1]
    invd = 1.0 / jnp.maximum(deg, 1.0)
    h = jnp.maximum((p_ref[0] + p_ref[1]) * invd + xr_ref[...], 0.0)
    o = jnp.dot(h, w_ref[...], preferred_element_type=jnp.float32)
    z_ref[...] = o[:, :H]
    xr2_ref[...] = o[:, H:] + b_ref[...]


def _k3_body(q_ref, dg_ref, xr_ref, wrow_ref, b_ref, out_ref):
    deg = dg_ref[0, :, :1] + dg_ref[1, :, :1]
    invd = 1.0 / jnp.maximum(deg, 1.0)
    h = jnp.maximum((q_ref[0] + q_ref[1]) * invd + xr_ref[...], 0.0)
    out_ref[...] = jnp.sum(h * wrow_ref[...], axis=1, keepdims=True) + b_ref[...]


_full = lambda *shape: pl.BlockSpec(shape, lambda m: tuple(0 for _ in shape))

_k1 = pl.pallas_call(
    _k1_body,
    grid=(_GRID,),
    in_specs=[
        pl.BlockSpec((_BM, D), lambda m: (m, 0)),
        _full(D, 2 * H),
        _full(1, H),
    ],
    out_specs=[pl.BlockSpec((_BM, H), lambda m: (m, 0))] * 2,
    out_shape=[jax.ShapeDtypeStruct((N, H), jnp.float32)] * 2,
)

_k2 = pl.pallas_call(
    _k2_body,
    grid=(_GRID,),
    in_specs=[
        pl.BlockSpec((NC, _BM, H), lambda m: (0, m, 0)),
        pl.BlockSpec((NC, _BM, 16), lambda m: (0, m, 0)),
        pl.BlockSpec((_BM, H), lambda m: (m, 0)),
        _full(H, 2 * H),
        _full(1, H),
    ],
    out_specs=[pl.BlockSpec((_BM, H), lambda m: (m, 0))] * 2,
    out_shape=[jax.ShapeDtypeStruct((N, H), jnp.float32)] * 2,
)

_k3 = pl.pallas_call(
    _k3_body,
    grid=(_GRID,),
    in_specs=[
        pl.BlockSpec((NC, _BM, H), lambda m: (0, m, 0)),
        pl.BlockSpec((NC, _BM, 16), lambda m: (0, m, 0)),
        pl.BlockSpec((_BM, H), lambda m: (m, 0)),
        _full(1, H),
        _full(1, 1),
    ],
    out_specs=pl.BlockSpec((_BM, 1), lambda m: (m, 0)),
    out_shape=jax.ShapeDtypeStruct((N, 1), jnp.float32),
)


def kernel(x, edge_index, W1l, W1r, b1, W2l, W2r, b2, Wout, bout):
    src = edge_index[0]
    dst = edge_index[1]
    pad = E_PAD - src.shape[0]
    ar = jnp.arange(pad, dtype=jnp.int32)
    srcp = jnp.concatenate([src, (ar * 97) % N]).reshape(NCHUNK, CH)
    dstp = jnp.concatenate([dst, N + ar % (N_ACC - N)]).reshape(NCHUNK, CH)

    w1 = jnp.concatenate([W1l.T, W1r.T], axis=1)
    w2 = jnp.concatenate([W2l.T, W2r.T], axis=1)
    b1r = b1.reshape(1, H)
    b2r = b2.reshape(1, H)
    wrow = Wout.reshape(1, H)
    br = bout.reshape(1, 1)

    degp = _sc_degree(dstp)
    z1, xr1 = _k1(x, w1, b1r)
    p1 = _sc_segsum(z1, srcp, dstp)
    z2, xr2 = _k2(p1, degp, xr1, w2, b2r)
    p2 = _sc_segsum(z2, srcp, dstp)
    out = _k3(p2, degp, xr2, wrow, br)
    return out[:, 0]


# speedup vs baseline: 14.4901x; 1.3812x over previous
import functools

import jax
import jax.numpy as jnp
from jax import lax
from jax.experimental import pallas as pl
from jax.experimental.pallas import tpu as pltpu
from jax.experimental.pallas import tpu_sc as plsc

N = 10000
D = 128
H = 64

NC = 2
NS = 16
CH = 128
CPT = 80
NCHUNK = NC * NS * CPT
E_PAD = NCHUNK * CH
N_ACC = 10240
RPT = N_ACC // NS

_vmesh = plsc.VectorSubcoreMesh(core_axis_name="c", subcore_axis_name="s")
_sc_params = pltpu.CompilerParams(use_tc_tiling_on_sc=False)


@functools.partial(
    pl.kernel,
    out_type=jax.ShapeDtypeStruct((NC, N_ACC, H), jnp.float32),
    mesh=_vmesh,
    scratch_types=[
        pltpu.VMEM((CPT, CH), jnp.int32),
        pltpu.VMEM((CPT, CH), jnp.int32),
        pltpu.VMEM((CH, H), jnp.float32),
        pltpu.VMEM((CH, H), jnp.float32),
        pltpu.VMEM_SHARED((N_ACC, H), jnp.float32),
        pltpu.SemaphoreType.DMA,
        pltpu.SemaphoreType.DMA,
    ],
    compiler_params=_sc_params,
)
def _sc_segsum(table_hbm, srcp_hbm, dstp_hbm, out_hbm,
               sidx, didx, rows, rows1, acc, sem, sem1):
    c = lax.axis_index("c")
    s = lax.axis_index("s")
    base_chunk = c * (NS * CPT) + s * CPT

    pltpu.sync_copy(srcp_hbm.at[pl.ds(base_chunk, CPT)], sidx)
    pltpu.sync_copy(dstp_hbm.at[pl.ds(base_chunk, CPT)], didx)

    @pl.loop(0, CH)
    def _(i):
        @pl.loop(0, H // 16)
        def _(j):
            rows[i, pl.ds(j * 16, 16)] = jnp.zeros((16,), jnp.float32)

    @pl.loop(0, RPT // CH)
    def _(j):
        pltpu.sync_copy(rows, acc.at[pl.ds(s * RPT + j * CH, CH)])

    plsc.subcore_barrier()

    pltpu.async_copy(table_hbm.at[sidx.at[0]], rows, sem)

    @pl.loop(0, CPT // 2)
    def _(g):
        i = g * 2
        pltpu.async_copy(table_hbm.at[sidx.at[i + 1]], rows1, sem1)
        pltpu.make_async_copy(table_hbm.at[sidx.at[i]], rows, sem).wait()
        pltpu.sync_copy(rows, acc.at[didx.at[i]], add=True)

        @pl.when(g < CPT // 2 - 1)
        def _():
            pltpu.async_copy(table_hbm.at[sidx.at[i + 2]], rows, sem)

        pltpu.make_async_copy(table_hbm.at[sidx.at[i + 1]], rows1, sem1).wait()
        pltpu.sync_copy(rows1, acc.at[didx.at[i + 1]], add=True)

    plsc.subcore_barrier()

    pltpu.sync_copy(acc.at[pl.ds(s * RPT, RPT)],
                    out_hbm.at[c].at[pl.ds(s * RPT, RPT)])


@functools.partial(
    pl.kernel,
    out_type=jax.ShapeDtypeStruct((NC, N_ACC, 16), jnp.float32),
    mesh=_vmesh,
    scratch_types=[
        pltpu.VMEM((CPT, CH), jnp.int32),
        pltpu.VMEM((CH, 16), jnp.float32),
        pltpu.VMEM_SHARED((N_ACC, 16), jnp.float32),
    ],
    compiler_params=_sc_params,
)
def _sc_degree(dstp_hbm, out_hbm, didx, ones, acc):
    c = lax.axis_index("c")
    s = lax.axis_index("s")
    base_chunk = c * (NS * CPT) + s * CPT

    pltpu.sync_copy(dstp_hbm.at[pl.ds(base_chunk, CPT)], didx)

    @pl.loop(0, CH)
    def _(i):
        ones[i, :] = jnp.zeros((16,), jnp.float32)

    @pl.loop(0, RPT // CH)
    def _(j):
        pltpu.sync_copy(ones, acc.at[pl.ds(s * RPT + j * CH, CH)])

    @pl.loop(0, CH)
    def _(i):
        ones[i, :] = jnp.ones((16,), jnp.float32)

    plsc.subcore_barrier()

    @pl.loop(0, CPT)
    def _(i):
        pltpu.sync_copy(ones, acc.at[didx.at[i]], add=True)

    plsc.subcore_barrier()

    pltpu.sync_copy(acc.at[pl.ds(s * RPT, RPT)],
                    out_hbm.at[c].at[pl.ds(s * RPT, RPT)])


_BM = 1000
_GRID = N // _BM


def _k1_body(x_ref, w_ref, b_ref, z_ref, xr_ref):
    o = jnp.dot(x_ref[...], w_ref[...], preferred_element_type=jnp.float32)
    z_ref[...] = o[:, :H]
    xr_ref[...] = o[:, H:] + b_ref[...]


def _k2_body(p_ref, dg_ref, xr_ref, w_ref, b_ref, z_ref, xr2_ref):
    deg = dg_ref[0, :, :1] + dg_ref[1, :, :1]
    invd = 1.0 / jnp.maximum(deg, 1.0)
    h = jnp.maximum((p_ref[0] + p_ref[1]) * invd + xr_ref[...], 0.0)
    o = jnp.dot(h, w_ref[...], preferred_element_type=jnp.float32)
    z_ref[...] = o[:, :H]
    xr2_ref[...] = o[:, H:] + b_ref[...]


def _k3_body(q_ref, dg_ref, xr_ref, wrow_ref, b_ref, out_ref):
    deg = dg_ref[0, :, :1] + dg_ref[1, :, :1]
    invd = 1.0 / jnp.maximum(deg, 1.0)
    h = jnp.maximum((q_ref[0] + q_ref[1]) * invd + xr_ref[...], 0.0)
    out_ref[...] = jnp.sum(h * wrow_ref[...], axis=1, keepdims=True) + b_ref[...]


_full = lambda *shape: pl.BlockSpec(shape, lambda m: tuple(0 for _ in shape))

_k1 = pl.pallas_call(
    _k1_body,
    grid=(_GRID,),
    in_specs=[
        pl.BlockSpec((_BM, D), lambda m: (m, 0)),
        _full(D, 2 * H),
        _full(1, H),
    ],
    out_specs=[pl.BlockSpec((_BM, H), lambda m: (m, 0))] * 2,
    out_shape=[jax.ShapeDtypeStruct((N, H), jnp.float32)] * 2,
)

_k2 = pl.pallas_call(
    _k2_body,
    grid=(_GRID,),
    in_specs=[
        pl.BlockSpec((NC, _BM, H), lambda m: (0, m, 0)),
        pl.BlockSpec((NC, _BM, 16), lambda m: (0, m, 0)),
        pl.BlockSpec((_BM, H), lambda m: (m, 0)),
        _full(H, 2 * H),
        _full(1, H),
    ],
    out_specs=[pl.BlockSpec((_BM, H), lambda m: (m, 0))] * 2,
    out_shape=[jax.ShapeDtypeStruct((N, H), jnp.float32)] * 2,
)

_k3 = pl.pallas_call(
    _k3_body,
    grid=(_GRID,),
    in_specs=[
        pl.BlockSpec((NC, _BM, H), lambda m: (0, m, 0)),
        pl.BlockSpec((NC, _BM, 16), lambda m: (0, m, 0)),
        pl.BlockSpec((_BM, H), lambda m: (m, 0)),
        _full(1, H),
        _full(1, 1),
    ],
    out_specs=pl.BlockSpec((_BM, 1), lambda m: (m, 0)),
    out_shape=jax.ShapeDtypeStruct((N, 1), jnp.float32),
)


def kernel(x, edge_index, W1l, W1r, b1, W2l, W2r, b2, Wout, bout):
    src = edge_index[0]
    dst = edge_index[1]
    pad = E_PAD - src.shape[0]
    ar = jnp.arange(pad, dtype=jnp.int32)
    srcp = jnp.concatenate([src, (ar * 97) % N]).reshape(NCHUNK, CH)
    dstp = jnp.concatenate([dst, N + ar % (N_ACC - N)]).reshape(NCHUNK, CH)

    w1 = jnp.concatenate([W1l.T, W1r.T], axis=1)
    w2 = jnp.concatenate([W2l.T, W2r.T], axis=1)
    b1r = b1.reshape(1, H)
    b2r = b2.reshape(1, H)
    wrow = Wout.reshape(1, H)
    br = bout.reshape(1, 1)

    degp = _sc_degree(dstp)
    z1, xr1 = _k1(x, w1, b1r)
    p1 = _sc_segsum(z1, srcp, dstp)
    z2, xr2 = _k2(p1, degp, xr1, w2, b2r)
    p2 = _sc_segsum(z2, srcp, dstp)
    out = _k3(p2, degp, xr2, wrow, br)
    return out[:, 0]
